# Initial kernel scaffold; baseline (speedup 1.0000x reference)
import jax
import jax.numpy as jnp
from jax.experimental import pallas as pl


def kernel(index_vec, neighbor_index_vec, len_sum, table):
    raise NotImplementedError("write your pallas kernel here")


# speedup vs baseline: 1.1083x; 1.1083x over previous
import jax
import jax.numpy as jnp
from jax.experimental import pallas as pl
from jax.experimental.pallas import tpu as pltpu


def _loss_kernel(idx_ref, a_ref, b_ref, len_ref, out_ref):
    d = jnp.abs(a_ref[0] - b_ref[0])
    col = jax.lax.broadcasted_iota(jnp.int32, (1, 64), 1)
    d = jnp.where(col < 50, d, jnp.float32(jnp.inf))
    min_d = jnp.min(d)
    ls = len_ref[0]
    l1 = jnp.abs(min_d - ls)
    out_ref[0] = jnp.where(min_d < ls, jnp.float32(100.0) * l1, l1)


def kernel(index_vec, neighbor_index_vec, len_sum, table):
    idx = jnp.stack([index_vec[0], neighbor_index_vec[0]]).astype(jnp.int32)
    len_arr = jnp.reshape(len_sum, (1,))
    table3 = table.reshape(table.shape[0], 1, table.shape[1])
    out = pl.pallas_call(
        _loss_kernel,
        grid_spec=pltpu.PrefetchScalarGridSpec(
            num_scalar_prefetch=1,
            grid=(1,),
            in_specs=[
                pl.BlockSpec((1, 1, 64), lambda i, idx_ref: (idx_ref[0], 0, 0)),
                pl.BlockSpec((1, 1, 64), lambda i, idx_ref: (idx_ref[1], 0, 0)),
                pl.BlockSpec(memory_space=pltpu.SMEM),
            ],
            out_specs=pl.BlockSpec(memory_space=pltpu.SMEM),
        ),
        out_shape=jax.ShapeDtypeStruct((1,), jnp.float32),
    )(idx, table3, table3, len_arr)
    return out[0]
